# Initial kernel scaffold; baseline (speedup 1.0000x reference)
import functools

import jax
import jax.numpy as jnp
import numpy as np
from jax import lax
from jax.experimental import pallas as pl
from jax.experimental.pallas import tpu as pltpu
from jax.experimental.pallas import tpu_sc as plsc

B, L, D = 1024, 20, 10000
VOCAB, NUM_CLASSES, NGRAM_N = 1000, 100, 3

NC, NS = 2, 16
NW = NC * NS
BPW = B // NW

NCHUNK = 5
DC = D // NCHUNK
HALO = NGRAM_N - 1
W = DC + 16

_NT = L - (NGRAM_N - 1)


def _build_haloed_table(id_weight):
    cols = (np.arange(W)[None, :] + (np.arange(NCHUNK) * DC)[:, None] - HALO) % D
    th = jnp.take(id_weight, jnp.asarray(cols.reshape(-1)), axis=1)
    return th.reshape(VOCAB * NCHUNK, W)


def _sc_encode(table_h, x):
    mesh = plsc.VectorSubcoreMesh(
        core_axis_name="c", subcore_axis_name="s", num_cores=NC, num_subcores=NS
    )

    @functools.partial(
        pl.kernel,
        out_type=jax.ShapeDtypeStruct((B, D), jnp.float32),
        mesh=mesh,
        scratch_types=[
            pltpu.VMEM((BPW, L), jnp.int32),
            pltpu.VMEM((L,), jnp.int32),
            pltpu.VMEM((L, W), jnp.float32),
            pltpu.VMEM((DC,), jnp.float32),
            pltpu.SemaphoreType.DMA,
        ],
    )
    def enc_kernel(table_hbm, x_hbm, enc_hbm, xw, idxv, buf, acc, sem):
        wid = lax.axis_index("s") * NC + lax.axis_index("c")
        base_b = wid * BPW
        pltpu.sync_copy(x_hbm.at[pl.ds(base_b, BPW)], xw)

        def body_b(i, carry):
            for c in range(NCHUNK):
                idxv[pl.ds(0, 16)] = xw[i, pl.ds(0, 16)] * NCHUNK + c
                idxv[pl.ds(4, 16)] = xw[i, pl.ds(4, 16)] * NCHUNK + c
                pltpu.async_copy(table_hbm.at[idxv], buf, sem).wait()

                def gbody(g, _):
                    base = g * 16
                    a = buf[0, pl.ds(base, 16)]
                    a = a * buf[1, pl.ds(base + 1, 16)]
                    a = a * buf[2, pl.ds(base + 2, 16)]
                    for t in range(1, _NT):
                        v = buf[t, pl.ds(base, 16)]
                        v = v * buf[t + 1, pl.ds(base + 1, 16)]
                        v = v * buf[t + 2, pl.ds(base + 2, 16)]
                        a = a + v
                    acc[pl.ds(base, 16)] = a
                    return _

                lax.fori_loop(0, DC // 16, gbody, 0)
                pltpu.sync_copy(acc, enc_hbm.at[base_b + i, pl.ds(c * DC, DC)])
            return carry

        lax.fori_loop(0, BPW, body_b, 0)

    return enc_kernel(table_h, x)


def _tc_search(enc, am_weight):
    BB = 128

    def body(enc_ref, am_ref, out_ref):
        am = am_ref[...]
        an = jnp.sqrt(jnp.sum(am * am, axis=1, keepdims=True)) + 1e-12
        am_n = am / an
        e = enc_ref[...]
        en = jnp.sqrt(jnp.sum(e * e, axis=1, keepdims=True)) + 1e-12
        s = lax.dot_general(
            e, am_n, (((1,), (1,)), ((), ())), preferred_element_type=jnp.float32
        )
        out_ref[...] = s / en

    return pl.pallas_call(
        body,
        grid=(B // BB,),
        in_specs=[
            pl.BlockSpec((BB, D), lambda i: (i, 0)),
            pl.BlockSpec((NUM_CLASSES, D), lambda i: (0, 0)),
        ],
        out_specs=pl.BlockSpec((BB, NUM_CLASSES), lambda i: (i, 0)),
        out_shape=jax.ShapeDtypeStruct((B, NUM_CLASSES), jnp.float32),
    )(enc, am_weight)


@jax.jit
def kernel(x, id_weight, am_weight):
    table_h = _build_haloed_table(id_weight)
    enc = _sc_encode(table_h, x.astype(jnp.int32))
    return _tc_search(enc, am_weight)


# speedup vs baseline: 1.6589x; 1.6589x over previous
import functools

import jax
import jax.numpy as jnp
import numpy as np
from jax import lax
from jax.experimental import pallas as pl
from jax.experimental.pallas import tpu as pltpu
from jax.experimental.pallas import tpu_sc as plsc

B, L, D = 1024, 20, 10000
VOCAB, NUM_CLASSES, NGRAM_N = 1000, 100, 3

NC, NS = 2, 16
NW = NC * NS
BPW = B // NW
RB = 8

NCHUNK = 5
DP = 10240
DC = DP // NCHUNK
HALO = NGRAM_N - 1
W = DC + 128

_NT = L - (NGRAM_N - 1)


def _build_haloed_table(id_weight):
    cols = np.arange(W)[None, :] + (np.arange(NCHUNK) * DC)[:, None] - HALO
    cols = np.where(cols < 0, cols + D, cols)
    wz = jnp.pad(id_weight, ((0, 0), (0, int(cols.max()) + 1 - D)))
    th = jnp.take(wz, jnp.asarray(cols.reshape(-1)), axis=1)
    return th.reshape(VOCAB * NCHUNK, W)


def _sc_encode(table_h, x):
    mesh = plsc.VectorSubcoreMesh(
        core_axis_name="c", subcore_axis_name="s", num_cores=NC, num_subcores=NS
    )

    @functools.partial(
        pl.kernel,
        out_type=jax.ShapeDtypeStruct((B, DP), jnp.float32),
        mesh=mesh,
        compiler_params=pltpu.CompilerParams(use_tc_tiling_on_sc=False),
        scratch_types=[
            pltpu.VMEM((BPW, L), jnp.int32),
            pltpu.VMEM((L,), jnp.int32),
            pltpu.VMEM((L, W), jnp.float32),
            pltpu.VMEM((RB, DC), jnp.float32),
            pltpu.SemaphoreType.DMA,
        ],
    )
    def enc_kernel(table_hbm, x_hbm, enc_hbm, xw, idxv, buf, acc, sem):
        wid = lax.axis_index("s") * NC + lax.axis_index("c")
        base_b = wid * BPW
        pltpu.sync_copy(x_hbm.at[pl.ds(base_b, BPW)], xw)

        def body_grp(i8, carry):
            def body_c(c, carry2):
                def body_r(r, carry3):
                    i = i8 * RB + r
                    idxv[pl.ds(0, 16)] = xw[i, pl.ds(0, 16)] * NCHUNK + c
                    idxv[pl.ds(4, 16)] = xw[i, pl.ds(4, 16)] * NCHUNK + c
                    pltpu.async_copy(table_hbm.at[idxv], buf, sem).wait()

                    for g in range(DC // 16):
                        base = g * 16

                        def tbody(t, a):
                            v = buf[t, pl.ds(base, 16)]
                            v = v * buf[t + 1, pl.ds(base + 1, 16)]
                            v = v * buf[t + 2, pl.ds(base + 2, 16)]
                            return a + v

                        acc[r, pl.ds(base, 16)] = lax.fori_loop(
                            0, _NT, tbody, jnp.zeros((16,), jnp.float32)
                        )
                    return carry3

                lax.fori_loop(0, RB, body_r, 0)
                row0 = pl.multiple_of(base_b + i8 * RB, RB)
                col0 = pl.multiple_of(c * DC, 256)
                pltpu.sync_copy(
                    acc, enc_hbm.at[pl.ds(row0, RB), pl.ds(col0, DC)]
                )
                return carry2

            lax.fori_loop(0, NCHUNK, body_c, 0)
            return carry

        lax.fori_loop(0, BPW // RB, body_grp, 0)

    return enc_kernel(table_h, x)


def _tc_search(enc, am_pad):
    BB = 128

    def body(enc_ref, am_ref, out_ref):
        am = am_ref[...]
        an = jnp.sqrt(jnp.sum(am * am, axis=1, keepdims=True)) + 1e-12
        am_n = am / an
        e = enc_ref[...]
        en = jnp.sqrt(jnp.sum(e * e, axis=1, keepdims=True)) + 1e-12
        s = lax.dot_general(
            e, am_n, (((1,), (1,)), ((), ())), preferred_element_type=jnp.float32
        )
        out_ref[...] = s / en

    return pl.pallas_call(
        body,
        grid=(B // BB,),
        in_specs=[
            pl.BlockSpec((BB, DP), lambda i: (i, 0)),
            pl.BlockSpec((NUM_CLASSES, DP), lambda i: (0, 0)),
        ],
        out_specs=pl.BlockSpec((BB, NUM_CLASSES), lambda i: (i, 0)),
        out_shape=jax.ShapeDtypeStruct((B, NUM_CLASSES), jnp.float32),
    )(enc, am_pad)


@jax.jit
def kernel(x, id_weight, am_weight):
    table_h = _build_haloed_table(id_weight)
    enc = _sc_encode(table_h, x.astype(jnp.int32))
    am_pad = jnp.pad(am_weight, ((0, 0), (0, DP - D)))
    return _tc_search(enc, am_pad)
